# Initial kernel scaffold; baseline (speedup 1.0000x reference)
import jax
import jax.numpy as jnp
from jax.experimental import pallas as pl


def kernel(params, input_ids):
    raise NotImplementedError("write your pallas kernel here")


# speedup vs baseline: 1.3974x; 1.3974x over previous
import functools

import numpy as np
import jax
import jax.numpy as jnp
from jax import lax
from jax.experimental import pallas as pl
from jax.experimental.pallas import tpu as pltpu
from jax.experimental.pallas import tpu_sc as plsc

D = 512
S = 2048
H = 16
HD = 64
RD = 64
CKV = 192
CQ = 192
NE = 8
TOPK = 2
FF = 4 * D
VOCAB = 16384

BLK = 256
NB_R = (S * TOPK + NE * BLK) // BLK
NPAD_R = NB_R * BLK


def _sc_gather(table, idx, chunk):
    nrows, d = idx.shape[0], table.shape[1]
    nw = 32
    b_per_w = nrows // nw
    nchunks = b_per_w // chunk
    mesh = plsc.VectorSubcoreMesh(core_axis_name="c", subcore_axis_name="s")

    @functools.partial(
        pl.kernel,
        mesh=mesh,
        out_type=jax.ShapeDtypeStruct((nrows, d), jnp.float32),
        scratch_types=[
            pltpu.VMEM((chunk,), jnp.int32),
            pltpu.VMEM((chunk, d), jnp.float32),
            pltpu.SemaphoreType.DMA,
        ],
    )
    def gk(table_hbm, idx_hbm, out_hbm, idx_v, rows_v, sem):
        wid = lax.axis_index("s") * 2 + lax.axis_index("c")
        for c in range(nchunks):
            base = wid * b_per_w + c * chunk
            pltpu.sync_copy(idx_hbm.at[pl.ds(base, chunk)], idx_v)
            pltpu.async_copy(table_hbm.at[idx_v], rows_v, sem).wait()
            pltpu.sync_copy(rows_v, out_hbm.at[pl.ds(base, chunk)])

    return gk(table, idx)


def _bdot(a, b):
    return jnp.dot(a.astype(jnp.bfloat16), b.astype(jnp.bfloat16),
                   preferred_element_type=jnp.float32)


def _ln_body(x_ref, g_ref, b_ref, o_ref):
    x = x_ref[...]
    m = jnp.mean(x, axis=-1, keepdims=True)
    v = jnp.mean((x - m) ** 2, axis=-1, keepdims=True)
    o_ref[...] = (x - m) / jnp.sqrt(v + 1e-5) * g_ref[...] + b_ref[...]


def _layernorm(x, p):
    return pl.pallas_call(
        _ln_body,
        out_shape=jax.ShapeDtypeStruct(x.shape, jnp.float32),
    )(x, p["g"].reshape(1, -1), p["b"].reshape(1, -1))


def _mm_body(x_ref, w_ref, b_ref, o_ref):
    o_ref[...] = _bdot(x_ref[...], w_ref[...]) + b_ref[...]


def _mm(x, w, b):
    m, kd = x.shape
    n = w.shape[1]
    nb = min(n, 1024)
    return pl.pallas_call(
        _mm_body,
        grid=(n // nb,),
        in_specs=[
            pl.BlockSpec((m, kd), lambda j: (0, 0)),
            pl.BlockSpec((kd, nb), lambda j: (0, j)),
            pl.BlockSpec((1, nb), lambda j: (0, j)),
        ],
        out_specs=pl.BlockSpec((m, nb), lambda j: (0, j)),
        out_shape=jax.ShapeDtypeStruct((m, n), jnp.float32),
    )(x, w, b.reshape(1, n))


def _rot(x, cos, sin):
    r1 = x[:, HD : HD + RD // 2]
    r2 = x[:, HD + RD // 2 :]
    return jnp.concatenate(
        [x[:, :HD], r1 * cos - r2 * sin, r1 * sin + r2 * cos], axis=1
    )


def _attn_body(q_ref, k_ref, v_ref, wo_ref, bo_ref, xres_ref, cos_ref, sin_ref,
               o_ref):
    h = pl.program_id(1)
    cos = cos_ref[0]
    sin = sin_ref[0]
    q = _rot(q_ref[...], cos, sin)
    k = _rot(k_ref[...], cos, sin)
    s = lax.dot_general(
        q.astype(jnp.bfloat16), k.astype(jnp.bfloat16),
        (((1,), (1,)), ((), ())), preferred_element_type=jnp.float32
    ) / np.sqrt(HD + RD)
    mx = jnp.max(s, axis=1, keepdims=True)
    e = jnp.exp(s - mx)
    probs = e / jnp.sum(e, axis=1, keepdims=True)
    ctx = _bdot(probs, v_ref[0])
    contrib = _bdot(ctx, wo_ref[0])

    @pl.when(h == 0)
    def _():
        o_ref[...] = xres_ref[...] + bo_ref[...]

    o_ref[...] += contrib


def _attention(q_all, k_all, v_heads, wo, bo, x_res):
    qb = 2
    qrows = S // qb
    cos, sin = _rope_tables()
    return pl.pallas_call(
        _attn_body,
        grid=(qb, H),
        in_specs=[
            pl.BlockSpec((qrows, HD + RD), lambda i, h: (i, h)),
            pl.BlockSpec((S, HD + RD), lambda i, h: (0, h)),
            pl.BlockSpec((1, S, HD), lambda i, h: (h, 0, 0)),
            pl.BlockSpec((1, HD, D), lambda i, h: (h, 0, 0)),
            pl.BlockSpec((1, D), lambda i, h: (0, 0)),
            pl.BlockSpec((qrows, D), lambda i, h: (i, 0)),
            pl.BlockSpec((1, 1, RD // 2), lambda i, h: (h, 0, 0)),
            pl.BlockSpec((1, 1, RD // 2), lambda i, h: (h, 0, 0)),
        ],
        out_specs=pl.BlockSpec((qrows, D), lambda i, h: (i, 0)),
        out_shape=jax.ShapeDtypeStruct((S, D), jnp.float32),
    )(q_all, k_all, v_heads, wo, bo.reshape(1, D), x_res,
      cos.reshape(H, 1, RD // 2), sin.reshape(H, 1, RD // 2))


def _router_body(xn_ref, wg_ref, bg_ref, ow_ref, oi_ref):
    s = _bdot(xn_ref[...], wg_ref[...]) + bg_ref[...]
    s = jax.nn.sigmoid(s)
    iota = lax.broadcasted_iota(jnp.int32, s.shape, 1)
    m1 = jnp.max(s, axis=1, keepdims=True)
    i1 = jnp.min(jnp.where(s == m1, iota, NE), axis=1, keepdims=True)
    s2 = jnp.where(iota == i1, -jnp.inf, s)
    m2 = jnp.max(s2, axis=1, keepdims=True)
    i2 = jnp.min(jnp.where(s2 == m2, iota, NE), axis=1, keepdims=True)
    ow_ref[...] = jnp.concatenate([m1, m2], axis=1)
    oi_ref[...] = jnp.concatenate([i1, i2], axis=1)


def _router(xn, mp):
    bg = (mp["gate"]["b"] + mp["bias"]).reshape(1, NE)
    return pl.pallas_call(
        _router_body,
        out_shape=[
            jax.ShapeDtypeStruct((S, TOPK), jnp.float32),
            jax.ShapeDtypeStruct((S, TOPK), jnp.int32),
        ],
    )(xn, mp["gate"]["W"], bg)


def _gelu(x):
    return 0.5 * x * (1.0 + lax.erf(x / np.sqrt(2.0).astype(np.float32)))


def _ffn_body(be_ref, x_ref, w1_ref, b1_ref, w2_ref, b2_ref, o_ref):
    hmid = _gelu(_bdot(x_ref[...], w1_ref[0]) + b1_ref[0])
    o_ref[...] = _bdot(hmid, w2_ref[0]) + b2_ref[0]


def _ffn_grouped(x_sorted, w1a, b1a, w2a, b2a, bexp):
    grid_spec = pltpu.PrefetchScalarGridSpec(
        num_scalar_prefetch=1,
        grid=(NB_R,),
        in_specs=[
            pl.BlockSpec((BLK, D), lambda b, be: (b, 0)),
            pl.BlockSpec((1, D, FF), lambda b, be: (be[b], 0, 0)),
            pl.BlockSpec((1, 1, FF), lambda b, be: (be[b], 0, 0)),
            pl.BlockSpec((1, FF, D), lambda b, be: (be[b], 0, 0)),
            pl.BlockSpec((1, 1, D), lambda b, be: (be[b], 0, 0)),
        ],
        out_specs=pl.BlockSpec((BLK, D), lambda b, be: (b, 0)),
    )
    return pl.pallas_call(
        _ffn_body,
        grid_spec=grid_spec,
        out_shape=jax.ShapeDtypeStruct((NPAD_R, D), jnp.float32),
    )(bexp, x_sorted, w1a, b1a, w2a, b2a)


def _shared_body(x_ref, w1_ref, b1_ref, w2_ref, b2_ref, o_ref):
    hmid = _gelu(_bdot(x_ref[...], w1_ref[...]) + b1_ref[...])
    o_ref[...] = _bdot(hmid, w2_ref[...]) + b2_ref[...]


def _shared_ffn(xn2, sp):
    rb = 4
    rows = S // rb
    return pl.pallas_call(
        _shared_body,
        grid=(rb,),
        in_specs=[
            pl.BlockSpec((rows, D), lambda i: (i, 0)),
            pl.BlockSpec((D, FF), lambda i: (0, 0)),
            pl.BlockSpec((1, FF), lambda i: (0, 0)),
            pl.BlockSpec((FF, D), lambda i: (0, 0)),
            pl.BlockSpec((1, D), lambda i: (0, 0)),
        ],
        out_specs=pl.BlockSpec((rows, D), lambda i: (i, 0)),
        out_shape=jax.ShapeDtypeStruct((S, D), jnp.float32),
    )(xn2, sp[0]["W"], sp[0]["b"].reshape(1, FF),
      sp[1]["W"], sp[1]["b"].reshape(1, D))


def _combine_body(xres_ref, ysh_ref, ypair_ref, w_ref, o_ref):
    w0 = w_ref[:, 0:1]
    w1 = w_ref[:, 1:2]
    comb = w0 * ypair_ref[:, :D] + w1 * ypair_ref[:, D:]
    o_ref[...] = xres_ref[...] + (ysh_ref[...] + comb)


def _combine(x_res, y_shared, y_pair, wts):
    return pl.pallas_call(
        _combine_body,
        out_shape=jax.ShapeDtypeStruct((S, D), jnp.float32),
    )(x_res, y_shared, y_pair, wts)


def _head_body(x_ref, w_ref, b_ref, o_ref):
    o_ref[...] = _bdot(x_ref[...], w_ref[...]) + b_ref[...]


def _head(xn, hp):
    nb = 16
    nc = VOCAB // nb
    return pl.pallas_call(
        _head_body,
        grid=(nb,),
        in_specs=[
            pl.BlockSpec((S, D), lambda j: (0, 0)),
            pl.BlockSpec((D, nc), lambda j: (0, j)),
            pl.BlockSpec((1, nc), lambda j: (0, j)),
        ],
        out_specs=pl.BlockSpec((S, nc), lambda j: (0, j)),
        out_shape=jax.ShapeDtypeStruct((S, VOCAB), jnp.float32),
    )(xn, hp["W"], hp["b"].reshape(1, VOCAB))


def _qk_weights(ap):
    wr = ap["qr"]["W"].reshape(-1, H, RD // 2, 2)
    br = ap["qr"]["b"].reshape(H, RD // 2, 2)
    wr2 = jnp.concatenate([wr[..., 0], wr[..., 1]], axis=-1)
    br2 = jnp.concatenate([br[..., 0], br[..., 1]], axis=-1)
    wq = jnp.concatenate(
        [ap["u_q"]["W"].reshape(CQ, H, HD), wr2], axis=-1
    ).reshape(CQ, H * (HD + RD))
    bq = jnp.concatenate(
        [ap["u_q"]["b"].reshape(H, HD), br2], axis=-1
    ).reshape(H * (HD + RD))
    wk = jnp.concatenate(
        [ap["u_k"]["W"].reshape(CKV, H, HD), wr2], axis=-1
    ).reshape(CKV, H * (HD + RD))
    bk = jnp.concatenate(
        [ap["u_k"]["b"].reshape(H, HD), br2], axis=-1
    ).reshape(H * (HD + RD))
    return wq, bq, wk, bk


def _rope_tables():
    pos = jnp.arange(H, dtype=jnp.float32)
    inv = 1.0 / (10000.0 ** (jnp.arange(0, RD, 2, dtype=jnp.float32) / RD))
    ang = pos[:, None] * inv[None, :]
    return jnp.cos(ang), jnp.sin(ang)


def _routing_indices(ti):
    a_e = ti.reshape(-1)
    onehot = (a_e[:, None] == jnp.arange(NE)[None, :]).astype(jnp.int32)
    pref = jnp.cumsum(onehot, axis=0)
    rank = jnp.take_along_axis(pref, a_e[:, None], axis=1)[:, 0] - 1
    counts = pref[-1]
    padded = ((counts + BLK - 1) // BLK) * BLK
    poff = jnp.concatenate(
        [jnp.zeros((1,), jnp.int32), jnp.cumsum(padded)[:-1].astype(jnp.int32)]
    )
    dest = (poff[a_e] + rank).astype(jnp.int32)
    tok = (jnp.arange(S * TOPK, dtype=jnp.int32) // TOPK)
    gidx = jnp.zeros((NPAD_R,), jnp.int32).at[dest].set(tok)
    bstart = jnp.arange(NB_R, dtype=jnp.int32) * BLK
    bexp = jnp.clip(
        jnp.searchsorted(poff, bstart, side="right") - 1, 0, NE - 1
    ).astype(jnp.int32)
    return gidx, dest, bexp


def _moe(mp, x, xn2):
    wts, ti = _router(xn2, mp)
    gidx, dest, bexp = _routing_indices(ti)
    w1a = jnp.stack([e[0]["W"] for e in mp["experts"]])
    b1a = jnp.stack([e[0]["b"] for e in mp["experts"]])
    w2a = jnp.stack([e[1]["W"] for e in mp["experts"]])
    b2a = jnp.stack([e[1]["b"] for e in mp["experts"]])
    swap = ti[:, 0] > ti[:, 1]
    wts = jnp.where(swap[:, None], wts[:, ::-1], wts)
    dest2 = dest.reshape(S, TOPK)
    dest = jnp.where(swap[:, None], dest2[:, ::-1], dest2).reshape(-1)
    x_sorted = _sc_gather(xn2, gidx, chunk=96)
    y_sorted = _ffn_grouped(
        x_sorted, w1a, b1a.reshape(NE, 1, FF), w2a, b2a.reshape(NE, 1, D), bexp
    )
    y_shared = _shared_ffn(xn2, mp["shared"])
    y_pair = _sc_gather(y_sorted, dest, chunk=128).reshape(S, 2 * D)
    return _combine(x, y_shared, y_pair, wts)


def _layer(lp, x):
    ap = lp["attn"]
    xn = _layernorm(x, lp["attn_norm"])
    wd = jnp.concatenate([ap["d_kv"]["W"], ap["d_q"]["W"]], axis=1)
    bd = jnp.concatenate([ap["d_kv"]["b"], ap["d_q"]["b"]])
    kvqc = _mm(xn, wd, bd)
    kv = kvqc[:, :CKV]
    qc = kvqc[:, CKV:]
    wq, bq, wk, bk = _qk_weights(ap)
    wkv = jnp.concatenate([wk, ap["u_v"]["W"]], axis=1)
    bkv = jnp.concatenate([bk, ap["u_v"]["b"]])
    kvout = _mm(kv, wkv, bkv)
    k_all = kvout[:, : H * (HD + RD)]
    v_flat = kvout[:, H * (HD + RD) :]
    q_all = _mm(qc, wq, bq)
    v_heads = v_flat.reshape(S, H, HD).transpose(1, 0, 2)
    wo = ap["out"]["W"].reshape(H, HD, D)
    x = _attention(q_all, k_all, v_heads, wo, ap["out"]["b"], x)
    xn2 = _layernorm(x, lp["moe_norm"])
    return _moe(lp["moe"], x, xn2)


def kernel(params, input_ids):
    ids = input_ids.reshape(-1).astype(jnp.int32)
    x = _sc_gather(params["embedding"], ids, chunk=64)
    for lp in params["layers"]:
        x = _layer(lp, x)
    xn = _layernorm(x, params["final_norm"])
    logits = _head(xn, params["head"])
    return logits.reshape(1, S, VOCAB)
